# baseline (speedup 1.0000x reference)
import functools

import jax
import jax.numpy as jnp
from jax import lax
from jax.experimental import pallas as pl
from jax.experimental.pallas import tpu as pltpu
from jax.experimental.pallas import tpu_sc as plsc

_N = 50000
_D = 512
_G = 128
_R = 80
_NB = _N // _R
_NC = 2
_NS = 16
_NW = _NC * _NS
_TRIPS = (_NB + _NW - 1) // _NW
_ACC = _G * _D


def _pool_sc(x_hbm, batch_hbm, part_hbm, rows_v, idx_v, acc2_v, ld_v,
             red_v, stage_sh):
    cid = lax.axis_index("c")
    sid = lax.axis_index("s")
    wid = cid * _NS + sid

    def _zbody(i, carry):
        for r in range(_G):
            acc2_v[r, pl.ds(i * 16, 16)] = jnp.zeros((16,), jnp.float32)
        return carry

    lax.fori_loop(0, _D // 16, _zbody, 0)

    lanes = lax.broadcasted_iota(jnp.int32, (16,), 0)

    def _body(t, carry):
        blk = wid + t * _NW

        @pl.when(blk < _NB)
        def _():
            base = blk * _R
            pltpu.sync_copy(batch_hbm.at[pl.ds(base, _R)], idx_v)
            pltpu.sync_copy(x_hbm.at[pl.ds(base * _D, _R * _D)], rows_v)

            def _grp(g, carry2):
                for rr in range(16):
                    seg_vec = plsc.load_gather(
                        idx_v, [jnp.full((16,), g * 16 + rr, jnp.int32)])
                    rbase = (g * 16 + rr) * _D
                    for c in range(_D // 16):
                        vals = rows_v[pl.ds(rbase + c * 16, 16)]
                        plsc.addupdate_scatter(acc2_v, [seg_vec,
                                                        lanes + (c * 16)],
                                               vals)
                return carry2

            lax.fori_loop(0, _R // 16, _grp, 0)

        return carry

    lax.fori_loop(0, _TRIPS, _body, 0)

    def _round(q, carry):
        pltpu.sync_copy(acc2_v.at[pl.ds(q * 16, 16)], stage_sh.at[sid])
        plsc.subcore_barrier()

        pltpu.sync_copy(stage_sh.at[0, sid], red_v)

        def _rbody(src, carry2):
            pltpu.sync_copy(stage_sh.at[src, sid], ld_v)

            def _abody(i, carry3):
                o = i * 16
                red_v[pl.ds(o, 16)] = red_v[pl.ds(o, 16)] + ld_v[pl.ds(o, 16)]
                return carry3

            lax.fori_loop(0, _D // 16, _abody, 0)
            return carry2

        lax.fori_loop(1, _NS, _rbody, 0)

        pltpu.sync_copy(red_v, part_hbm.at[cid * _G + q * 16 + sid])
        plsc.subcore_barrier()
        return carry

    lax.fori_loop(0, _G // 16, _round, 0)


_pool = pl.kernel(
    _pool_sc,
    out_type=jax.ShapeDtypeStruct((_NC * _G, _D), jnp.float32),
    mesh=plsc.VectorSubcoreMesh(core_axis_name="c", subcore_axis_name="s"),
    compiler_params=pltpu.CompilerParams(use_tc_tiling_on_sc=False,
                                         needs_layout_passes=False),
    scratch_types=[
        pltpu.VMEM((_R * _D,), jnp.float32),
        pltpu.VMEM((_R,), jnp.int32),
        pltpu.VMEM((_G, _D), jnp.float32),
        pltpu.VMEM((_D,), jnp.float32),
        pltpu.VMEM((_D,), jnp.float32),
        pltpu.VMEM_SHARED((_NS, 16, _D), jnp.float32),
    ],
)


def _proj_body(part_ref, w_ref, b_ref, out_ref):
    pooled = part_ref[0] + part_ref[1]
    out_ref[...] = jax.lax.dot_general(
        pooled, w_ref[...],
        dimension_numbers=(((1,), (1,)), ((), ())),
        preferred_element_type=jnp.float32) + b_ref[...]


@jax.jit
def kernel(x_0, batch_0, W, b):
    partials = _pool(x_0.reshape(_N * _D), batch_0)
    logits = pl.pallas_call(
        _proj_body,
        out_shape=jax.ShapeDtypeStruct((_G, _D), jnp.float32),
    )(partials.reshape(_NC, _G, _D), W, b.reshape(1, _D))
    return logits


# speedup vs baseline: 1.0010x; 1.0010x over previous
import functools

import jax
import jax.numpy as jnp
from jax import lax
from jax.experimental import pallas as pl
from jax.experimental.pallas import tpu as pltpu
from jax.experimental.pallas import tpu_sc as plsc

_N = 50000
_D = 512
_G = 128
_R = 80
_NB = _N // _R
_NC = 2
_NS = 16
_NW = _NC * _NS
_TRIPS = (_NB + _NW - 1) // _NW
_ACC = _G * _D


def _pool_sc(x_hbm, batch_hbm, part_hbm, rows_v, idx_v, acc2_v, ld_v,
             red_v, stage_sh):
    cid = lax.axis_index("c")
    sid = lax.axis_index("s")
    wid = cid * _NS + sid

    def _zbody(i, carry):
        for r in range(_G):
            acc2_v[r, pl.ds(i * 16, 16)] = jnp.zeros((16,), jnp.float32)
        return carry

    lax.fori_loop(0, _D // 16, _zbody, 0)

    lanes = lax.broadcasted_iota(jnp.int32, (16,), 0)

    def _body(t, carry):
        blk = wid + t * _NW

        @pl.when(blk < _NB)
        def _():
            base = blk * _R
            pltpu.sync_copy(batch_hbm.at[pl.ds(base, _R)], idx_v)
            pltpu.sync_copy(x_hbm.at[pl.ds(base, _R)], rows_v)

            def _grp(g, carry2):
                for rr in range(16):
                    seg_vec = plsc.load_gather(
                        idx_v, [jnp.full((16,), g * 16 + rr, jnp.int32)])
                    r = g * 16 + rr
                    for c in range(_D // 16):
                        vals = rows_v[r, pl.ds(c * 16, 16)]
                        plsc.addupdate_scatter(acc2_v, [seg_vec,
                                                        lanes + (c * 16)],
                                               vals)
                return carry2

            lax.fori_loop(0, _R // 16, _grp, 0)

        return carry

    lax.fori_loop(0, _TRIPS, _body, 0)

    def _round(q, carry):
        pltpu.sync_copy(acc2_v.at[pl.ds(q * 16, 16)], stage_sh.at[sid])
        plsc.subcore_barrier()

        pltpu.sync_copy(stage_sh.at[0, sid], red_v)

        def _rbody(src, carry2):
            pltpu.sync_copy(stage_sh.at[src, sid], ld_v)

            def _abody(i, carry3):
                o = i * 16
                red_v[pl.ds(o, 16)] = red_v[pl.ds(o, 16)] + ld_v[pl.ds(o, 16)]
                return carry3

            lax.fori_loop(0, _D // 16, _abody, 0)
            return carry2

        lax.fori_loop(1, _NS, _rbody, 0)

        pltpu.sync_copy(red_v, part_hbm.at[cid * _G + q * 16 + sid])
        plsc.subcore_barrier()
        return carry

    lax.fori_loop(0, _G // 16, _round, 0)


_pool = pl.kernel(
    _pool_sc,
    out_type=jax.ShapeDtypeStruct((_NC * _G, _D), jnp.float32),
    mesh=plsc.VectorSubcoreMesh(core_axis_name="c", subcore_axis_name="s"),
    compiler_params=pltpu.CompilerParams(use_tc_tiling_on_sc=False,
                                         needs_layout_passes=False),
    scratch_types=[
        pltpu.VMEM((_R, _D), jnp.float32),
        pltpu.VMEM((_R,), jnp.int32),
        pltpu.VMEM((_G, _D), jnp.float32),
        pltpu.VMEM((_D,), jnp.float32),
        pltpu.VMEM((_D,), jnp.float32),
        pltpu.VMEM_SHARED((_NS, 16, _D), jnp.float32),
    ],
)


def _proj_body(part_ref, w_ref, b_ref, out_ref):
    pooled = part_ref[0] + part_ref[1]
    out_ref[...] = jax.lax.dot_general(
        pooled, w_ref[...],
        dimension_numbers=(((1,), (1,)), ((), ())),
        preferred_element_type=jnp.float32) + b_ref[...]


@jax.jit
def kernel(x_0, batch_0, W, b):
    partials = _pool(x_0, batch_0)
    logits = pl.pallas_call(
        _proj_body,
        out_shape=jax.ShapeDtypeStruct((_G, _D), jnp.float32),
    )(partials.reshape(_NC, _G, _D), W, b.reshape(1, _D))
    return logits


# speedup vs baseline: 1.4533x; 1.4518x over previous
import functools

import jax
import jax.numpy as jnp
from jax import lax
from jax.experimental import pallas as pl
from jax.experimental.pallas import tpu as pltpu
from jax.experimental.pallas import tpu_sc as plsc

_N = 50000
_D = 512
_G = 128
_R = 80
_NB = _N // _R
_NC = 2
_NS = 16
_NW = _NC * _NS
_TRIPS = (_NB + _NW - 1) // _NW
_ACC = _G * _D


def _pool_sc(x_hbm, batch_hbm, part_hbm, rows_v, idx_v, acc2_v, ld_v,
             red_v, stage_sh):
    cid = lax.axis_index("c")
    sid = lax.axis_index("s")
    wid = cid * _NS + sid

    def _zbody(i, carry):
        for r in range(_G):
            acc2_v[r, pl.ds(i * 16, 16)] = jnp.zeros((16,), jnp.float32)
        return carry

    lax.fori_loop(0, _D // 16, _zbody, 0)

    lanes = lax.broadcasted_iota(jnp.int32, (16,), 0)

    def _body(t, carry):
        blk = wid + t * _NW

        @pl.when(blk < _NB)
        def _():
            base = blk * _R
            pltpu.sync_copy(batch_hbm.at[pl.ds(base, _R)], idx_v)
            pltpu.sync_copy(x_hbm.at[pl.ds(base, _R)], rows_v)

            def _grp(g, carry2):
                for rr in range(16):
                    seg_vec = plsc.load_gather(
                        idx_v, [jnp.full((16,), g * 16 + rr, jnp.int32)])
                    r = g * 16 + rr

                    @plsc.parallel_loop(0, _D, 16, unroll=8)
                    def _strip(c):
                        vals = rows_v[r, pl.ds(c, 16)]
                        plsc.addupdate_scatter(acc2_v, [seg_vec, lanes + c],
                                               vals)
                return carry2

            lax.fori_loop(0, _R // 16, _grp, 0)

        return carry

    lax.fori_loop(0, _TRIPS, _body, 0)

    def _round(q, carry):
        pltpu.sync_copy(acc2_v.at[pl.ds(q * 16, 16)], stage_sh.at[sid])
        plsc.subcore_barrier()

        pltpu.sync_copy(stage_sh.at[0, sid], red_v)

        def _rbody(src, carry2):
            pltpu.sync_copy(stage_sh.at[src, sid], ld_v)

            def _abody(i, carry3):
                o = i * 16
                red_v[pl.ds(o, 16)] = red_v[pl.ds(o, 16)] + ld_v[pl.ds(o, 16)]
                return carry3

            lax.fori_loop(0, _D // 16, _abody, 0)
            return carry2

        lax.fori_loop(1, _NS, _rbody, 0)

        pltpu.sync_copy(red_v, part_hbm.at[cid * _G + q * 16 + sid])
        plsc.subcore_barrier()
        return carry

    lax.fori_loop(0, _G // 16, _round, 0)


_pool = pl.kernel(
    _pool_sc,
    out_type=jax.ShapeDtypeStruct((_NC * _G, _D), jnp.float32),
    mesh=plsc.VectorSubcoreMesh(core_axis_name="c", subcore_axis_name="s"),
    compiler_params=pltpu.CompilerParams(use_tc_tiling_on_sc=False,
                                         needs_layout_passes=False),
    scratch_types=[
        pltpu.VMEM((_R, _D), jnp.float32),
        pltpu.VMEM((_R,), jnp.int32),
        pltpu.VMEM((_G, _D), jnp.float32),
        pltpu.VMEM((_D,), jnp.float32),
        pltpu.VMEM((_D,), jnp.float32),
        pltpu.VMEM_SHARED((_NS, 16, _D), jnp.float32),
    ],
)


def _proj_body(part_ref, w_ref, b_ref, out_ref):
    pooled = part_ref[0] + part_ref[1]
    out_ref[...] = jax.lax.dot_general(
        pooled, w_ref[...],
        dimension_numbers=(((1,), (1,)), ((), ())),
        preferred_element_type=jnp.float32) + b_ref[...]


@jax.jit
def kernel(x_0, batch_0, W, b):
    partials = _pool(x_0, batch_0)
    logits = pl.pallas_call(
        _proj_body,
        out_shape=jax.ShapeDtypeStruct((_G, _D), jnp.float32),
    )(partials.reshape(_NC, _G, _D), W, b.reshape(1, _D))
    return logits


# speedup vs baseline: 1.9874x; 1.3675x over previous
import functools

import jax
import jax.numpy as jnp
from jax import lax
from jax.experimental import pallas as pl
from jax.experimental.pallas import tpu as pltpu
from jax.experimental.pallas import tpu_sc as plsc

_N = 50000
_D = 512
_G = 128
_R = 80
_NB = _N // _R
_NC = 2
_NS = 16
_NW = _NC * _NS
_TRIPS = (_NB + _NW - 1) // _NW
_ACC = _G * _D


def _pool_sc(x_hbm, batch_hbm, part_hbm, rows_v, idx_v, acc2_v, ld_v,
             red_v, stage_sh):
    cid = lax.axis_index("c")
    sid = lax.axis_index("s")
    wid = cid * _NS + sid

    def _zbody(i, carry):
        for r in range(_G):
            acc2_v[r, pl.ds(i * 16, 16)] = jnp.zeros((16,), jnp.float32)
        return carry

    lax.fori_loop(0, _D // 16, _zbody, 0)

    lanes = lax.broadcasted_iota(jnp.int32, (16,), 0)

    def _body(t, carry):
        blk = wid + t * _NW

        @pl.when(blk < _NB)
        def _():
            base = blk * _R
            pltpu.sync_copy(batch_hbm.at[pl.ds(base, _R)], idx_v)
            pltpu.sync_copy(x_hbm.at[pl.ds(base, _R)], rows_v)

            def _grp(g, carry2):
                for rr in range(16):
                    seg_vec = plsc.load_gather(
                        idx_v, [jnp.full((16,), g * 16 + rr, jnp.int32)])
                    r = g * 16 + rr

                    @plsc.parallel_loop(0, _D, 16, unroll=8)
                    def _strip(c):
                        vals = rows_v[r, pl.ds(c, 16)]
                        plsc.addupdate_scatter(acc2_v, [seg_vec, lanes + c],
                                               vals)
                return carry2

            lax.fori_loop(0, _R // 16, _grp, 0)

        return carry

    lax.fori_loop(0, _TRIPS, _body, 0)

    def _round(q, carry):
        pltpu.sync_copy(acc2_v.at[pl.ds(q * 16, 16)], stage_sh.at[sid])
        plsc.subcore_barrier()

        pltpu.sync_copy(stage_sh.at[0, sid], red_v)

        def _rbody(src, carry2):
            pltpu.sync_copy(stage_sh.at[src, sid], ld_v)

            def _abody(i, carry3):
                o = i * 16
                red_v[pl.ds(o, 16)] = red_v[pl.ds(o, 16)] + ld_v[pl.ds(o, 16)]
                return carry3

            lax.fori_loop(0, _D // 16, _abody, 0)
            return carry2

        lax.fori_loop(1, _NS, _rbody, 0)

        pltpu.sync_copy(red_v, part_hbm.at[cid * _G + q * 16 + sid])
        plsc.subcore_barrier()
        return carry

    lax.fori_loop(0, _G // 16, _round, 0)


_pool = pl.kernel(
    _pool_sc,
    out_type=jax.ShapeDtypeStruct((_NC * _G, _D), jnp.float32),
    mesh=plsc.VectorSubcoreMesh(core_axis_name="c", subcore_axis_name="s"),
    compiler_params=pltpu.CompilerParams(use_tc_tiling_on_sc=True,
                                         needs_layout_passes=False),
    scratch_types=[
        pltpu.VMEM((_R, _D), jnp.float32),
        pltpu.VMEM((_R,), jnp.int32),
        pltpu.VMEM((_G, _D), jnp.float32),
        pltpu.VMEM((_D,), jnp.float32),
        pltpu.VMEM((_D,), jnp.float32),
        pltpu.VMEM_SHARED((_NS, 16, _D), jnp.float32),
    ],
)


def _proj_body(part_ref, w_ref, b_ref, out_ref):
    pooled = part_ref[0] + part_ref[1]
    out_ref[...] = jax.lax.dot_general(
        pooled, w_ref[...],
        dimension_numbers=(((1,), (1,)), ((), ())),
        preferred_element_type=jnp.float32) + b_ref[...]


@jax.jit
def kernel(x_0, batch_0, W, b):
    partials = _pool(x_0, batch_0)
    logits = pl.pallas_call(
        _proj_body,
        out_shape=jax.ShapeDtypeStruct((_G, _D), jnp.float32),
    )(partials.reshape(_NC, _G, _D), W, b.reshape(1, _D))
    return logits


# speedup vs baseline: 2.4477x; 1.2316x over previous
import functools

import jax
import jax.numpy as jnp
from jax import lax
from jax.experimental import pallas as pl
from jax.experimental.pallas import tpu as pltpu
from jax.experimental.pallas import tpu_sc as plsc

_N = 50000
_D = 512
_G = 128
_R = 80
_NB = _N // _R
_NC = 2
_NS = 16
_NW = _NC * _NS
_TRIPS = (_NB + _NW - 1) // _NW
_ACC = _G * _D


def _pool_sc(x_hbm, batch_hbm, part_hbm, rows_v, idx_v, acc2_v, ld_v,
             red_v, stage_sh):
    cid = lax.axis_index("c")
    sid = lax.axis_index("s")
    wid = cid * _NS + sid

    def _zbody(i, carry):
        for r in range(_G):
            acc2_v[r, pl.ds(i * 16, 16)] = jnp.zeros((16,), jnp.float32)
        return carry

    lax.fori_loop(0, _D // 16, _zbody, 0)

    lanes = lax.broadcasted_iota(jnp.int32, (16,), 0)

    def _body(t, carry):
        blk = wid + t * _NW

        @pl.when(blk < _NB)
        def _():
            base = blk * _R
            pltpu.sync_copy(batch_hbm.at[pl.ds(base, _R)], idx_v)
            pltpu.sync_copy(x_hbm.at[pl.ds(base, _R)], rows_v)

            def _grp(g, carry2):
                ids16 = idx_v[pl.ds(g * 16, 16)]
                first = ids16[0]
                last = ids16[15]

                @pl.when(first == last)
                def _fast():
                    seg_vec = plsc.load_gather(
                        idx_v, [jnp.full((16,), g * 16, jnp.int32)])

                    @plsc.parallel_loop(0, _D, 16, unroll=2)
                    def _strip(c):
                        s = rows_v[g * 16, pl.ds(c, 16)]
                        for rr in range(1, 16):
                            s = s + rows_v[g * 16 + rr, pl.ds(c, 16)]
                        plsc.addupdate_scatter(acc2_v, [seg_vec, lanes + c],
                                               s)

                @pl.when(first != last)
                def _slow():
                    for rr in range(16):
                        seg_vec = plsc.load_gather(
                            idx_v, [jnp.full((16,), g * 16 + rr, jnp.int32)])
                        r = g * 16 + rr

                        @plsc.parallel_loop(0, _D, 16, unroll=8)
                        def _strip(c):
                            vals = rows_v[r, pl.ds(c, 16)]
                            plsc.addupdate_scatter(acc2_v,
                                                   [seg_vec, lanes + c],
                                                   vals)

                return carry2

            lax.fori_loop(0, _R // 16, _grp, 0)

        return carry

    lax.fori_loop(0, _TRIPS, _body, 0)

    def _round(q, carry):
        pltpu.sync_copy(acc2_v.at[pl.ds(q * 16, 16)], stage_sh.at[sid])
        plsc.subcore_barrier()

        pltpu.sync_copy(stage_sh.at[0, sid], red_v)

        def _rbody(src, carry2):
            pltpu.sync_copy(stage_sh.at[src, sid], ld_v)

            def _abody(i, carry3):
                o = i * 16
                red_v[pl.ds(o, 16)] = red_v[pl.ds(o, 16)] + ld_v[pl.ds(o, 16)]
                return carry3

            lax.fori_loop(0, _D // 16, _abody, 0)
            return carry2

        lax.fori_loop(1, _NS, _rbody, 0)

        pltpu.sync_copy(red_v, part_hbm.at[cid * _G + q * 16 + sid])
        plsc.subcore_barrier()
        return carry

    lax.fori_loop(0, _G // 16, _round, 0)


_pool = pl.kernel(
    _pool_sc,
    out_type=jax.ShapeDtypeStruct((_NC * _G, _D), jnp.float32),
    mesh=plsc.VectorSubcoreMesh(core_axis_name="c", subcore_axis_name="s"),
    compiler_params=pltpu.CompilerParams(use_tc_tiling_on_sc=True,
                                         needs_layout_passes=False),
    scratch_types=[
        pltpu.VMEM((_R, _D), jnp.float32),
        pltpu.VMEM((_R,), jnp.int32),
        pltpu.VMEM((_G, _D), jnp.float32),
        pltpu.VMEM((_D,), jnp.float32),
        pltpu.VMEM((_D,), jnp.float32),
        pltpu.VMEM_SHARED((_NS, 16, _D), jnp.float32),
    ],
)


def _proj_body(part_ref, w_ref, b_ref, out_ref):
    pooled = part_ref[0] + part_ref[1]
    out_ref[...] = jax.lax.dot_general(
        pooled, w_ref[...],
        dimension_numbers=(((1,), (1,)), ((), ())),
        preferred_element_type=jnp.float32) + b_ref[...]


@jax.jit
def kernel(x_0, batch_0, W, b):
    partials = _pool(x_0, batch_0)
    logits = pl.pallas_call(
        _proj_body,
        out_shape=jax.ShapeDtypeStruct((_G, _D), jnp.float32),
    )(partials.reshape(_NC, _G, _D), W, b.reshape(1, _D))
    return logits
